# baseline (speedup 1.0000x reference)
import functools

import jax
import jax.numpy as jnp
from jax import lax
from jax.experimental import pallas as pl
from jax.experimental.pallas import tpu as pltpu
from jax.experimental.pallas import tpu_sc as plsc

_NBUF = 5
_AHEAD = 3
_IBUF = 5
_TB = 256


@functools.lru_cache(maxsize=None)
def _build(batch, hist, n_vocab, d):
    info = plsc.get_sparse_core_info()
    num_cores, num_subcores = info.num_cores, info.num_subcores
    num_workers = num_cores * num_subcores
    n_bb = batch // _TB
    bb_per_w = n_bb // num_workers
    assert batch % _TB == 0 and n_bb % num_workers == 0
    n_slots = hist * bb_per_w
    assert n_slots % _IBUF == 0 and _IBUF > _AHEAD and _NBUF >= _AHEAD + 2
    n_outer = n_slots // _IBUF
    tail = _NBUF - _AHEAD

    mesh = plsc.VectorSubcoreMesh(core_axis_name="c", subcore_axis_name="s")

    @functools.partial(
        pl.kernel,
        mesh=mesh,
        out_type=jax.ShapeDtypeStruct((n_bb, _TB, hist, 2 * d), jnp.float32),
        scratch_types=[
            pltpu.VMEM((_IBUF, _TB), jnp.int32),
            pltpu.VMEM((_NBUF, _TB, d), jnp.float32),
            [pltpu.SemaphoreType.DMA] * _IBUF,
            [pltpu.SemaphoreType.DMA] * _NBUF,
            [pltpu.SemaphoreType.DMA] * _NBUF,
        ],
        compiler_params=pltpu.CompilerParams(use_tc_tiling_on_sc=False),
    )
    def k(idx_hbm, tab_hbm, out_hbm, idx_v, rows_v, isem, gsem, osem):
        wid = lax.axis_index("s") * num_cores + lax.axis_index("c")
        bb0 = wid * bb_per_w

        def decode(g):
            l = g // bb_per_w
            return l, bb0 + (g - l * bb_per_w)

        def idx_ref(g):
            l, bb = decode(g)
            return idx_hbm.at[l, pl.ds(bb * _TB, _TB)]

        def idx_fetch(islot, g):
            pltpu.async_copy(idx_ref(g), idx_v.at[islot], isem[islot])

        def fire(slot, islot, g):
            pltpu.make_async_copy(idx_ref(g), idx_v.at[islot], isem[islot]).wait()
            pltpu.async_copy(tab_hbm.at[idx_v.at[islot]], rows_v.at[slot], gsem[slot])

        def drain(slot, islot):
            pltpu.make_async_copy(
                tab_hbm.at[idx_v.at[islot]], rows_v.at[slot], gsem[slot]
            ).wait()

        def out_ref(g):
            l, bb = decode(g)
            return out_hbm.at[bb, pl.ds(0, _TB), l, pl.ds(0, d)]

        def out_copy(slot, g):
            pltpu.async_copy(rows_v.at[slot], out_ref(g), osem[slot])

        def wait_out(slot, g):
            pltpu.make_async_copy(rows_v.at[slot], out_ref(g), osem[slot]).wait()

        for g0 in range(_IBUF):
            idx_fetch(g0, g0)
        for g0 in range(_AHEAD):
            fire(g0 % _NBUF, g0, g0)

        def outer(go, _):
            for b2 in range(_IBUF):
                g = go * _IBUF + b2
                slot = b2 % _NBUF
                fslot = (b2 + _AHEAD) % _NBUF
                if b2 < tail:
                    @pl.when(go >= 1)
                    def _():
                        wait_out(fslot, g + _AHEAD - _NBUF)
                else:
                    wait_out(fslot, g + _AHEAD - _NBUF)
                if b2 < _IBUF - _AHEAD:
                    fire(fslot, (b2 + _AHEAD) % _IBUF, g + _AHEAD)
                else:
                    @pl.when(go < n_outer - 1)
                    def _():
                        fire(fslot, (b2 + _AHEAD) % _IBUF, g + _AHEAD)
                drain(slot, b2)
                out_copy(slot, g)
                @pl.when(go < n_outer - 1)
                def _():
                    idx_fetch(b2, g + _IBUF)
            return ()

        lax.fori_loop(0, n_outer, outer, ())

        for i in range(tail):
            g_last = n_slots - tail + i
            wait_out(g_last % _NBUF, g_last)

    return k


def kernel(indices, table):
    b, h = indices.shape
    v, d = table.shape
    idx2 = jnp.transpose(indices.astype(jnp.int32))
    out = _build(b, h, v, d)(idx2, table)
    return out[:, :, :, :d].reshape(b, h, d)


# speedup vs baseline: 1.0089x; 1.0089x over previous
import functools

import jax
import jax.numpy as jnp
from jax import lax
from jax.experimental import pallas as pl
from jax.experimental.pallas import tpu as pltpu
from jax.experimental.pallas import tpu_sc as plsc

_SROWS = 2
_NBUF = 5
_AHEAD = 3
_IBUF = 2 * _NBUF
_TL = 8
_TB = 128


@functools.lru_cache(maxsize=None)
def _build(batch, hist, n_vocab, d):
    info = plsc.get_sparse_core_info()
    num_cores, num_subcores = info.num_cores, info.num_subcores
    num_workers = num_cores * num_subcores
    n_lb = hist // _TL
    n_bb = batch // _TB
    bb_per_w = n_bb // num_workers
    assert hist % _TL == 0 and batch % _TB == 0 and n_bb % num_workers == 0
    assert _TL % _SROWS == 0
    subs = _TL // _SROWS
    n_slots = n_lb * bb_per_w * subs
    assert n_slots % _IBUF == 0
    n_outer = n_slots // _IBUF
    tail = _NBUF - _AHEAD

    mesh = plsc.VectorSubcoreMesh(core_axis_name="c", subcore_axis_name="s")

    @functools.partial(
        pl.kernel,
        mesh=mesh,
        out_type=jax.ShapeDtypeStruct((n_bb, _TB, hist, 2 * d), jnp.float32),
        scratch_types=[
            pltpu.VMEM((_IBUF, _SROWS, _TB), jnp.int32),
            pltpu.VMEM((_NBUF, _SROWS, _TB, d), jnp.float32),
            [pltpu.SemaphoreType.DMA] * _IBUF,
            [pltpu.SemaphoreType.DMA] * _NBUF,
            [pltpu.SemaphoreType.DMA] * _NBUF,
        ],
        compiler_params=pltpu.CompilerParams(use_tc_tiling_on_sc=False),
    )
    def k(idx_hbm, tab_hbm, out_hbm, idx_v, rows_v, isem, gsem, osem):
        wid = lax.axis_index("s") * num_cores + lax.axis_index("c")
        bb0 = wid * bb_per_w

        def decode(g):
            c = g // subs
            sub = g - c * subs
            lb = c // bb_per_w
            bb = bb0 + (c - lb * bb_per_w)
            return lb, bb, sub * _SROWS

        def idx_fetch(islot, g):
            lb, bb, s0 = decode(g)
            pltpu.async_copy(
                idx_hbm.at[lb, bb, pl.ds(s0, _SROWS)], idx_v.at[islot], isem[islot]
            )

        def idx_wait(islot, g):
            lb, bb, s0 = decode(g)
            pltpu.make_async_copy(
                idx_hbm.at[lb, bb, pl.ds(s0, _SROWS)], idx_v.at[islot], isem[islot]
            ).wait()

        def fire(slot, islot, g):
            idx_wait(islot, g)
            for r in range(_SROWS):
                pltpu.async_copy(
                    tab_hbm.at[idx_v.at[islot, r]],
                    rows_v.at[slot, r],
                    gsem[slot],
                )

        def drain(slot, islot):
            for r in range(_SROWS):
                pltpu.make_async_copy(
                    tab_hbm.at[idx_v.at[islot, r]],
                    rows_v.at[slot, r],
                    gsem[slot],
                ).wait()

        def out_addr(g, r):
            lb, bb, s0 = decode(g)
            return out_hbm.at[bb, pl.ds(0, _TB), lb * _TL + s0 + r, pl.ds(0, d)]

        def out_copy(slot, g):
            for r in range(_SROWS):
                pltpu.async_copy(rows_v.at[slot, r], out_addr(g, r), osem[slot])

        def wait_out(slot, g):
            for r in range(_SROWS):
                pltpu.make_async_copy(
                    rows_v.at[slot, r], out_addr(g, r), osem[slot]
                ).wait()

        for g0 in range(2 * _AHEAD):
            idx_fetch(g0, g0)
        for g0 in range(_AHEAD):
            fire(g0 % _NBUF, g0, g0)

        def outer(go, _):
            for b2 in range(_IBUF):
                g = go * _IBUF + b2
                slot = b2 % _NBUF
                fslot = (b2 + _AHEAD) % _NBUF
                if b2 < _IBUF - 2 * _AHEAD:
                    idx_fetch((b2 + 2 * _AHEAD) % _IBUF, g + 2 * _AHEAD)
                else:
                    @pl.when(go < n_outer - 1)
                    def _():
                        idx_fetch((b2 + 2 * _AHEAD) % _IBUF, g + 2 * _AHEAD)
                if b2 < tail:
                    @pl.when(go >= 1)
                    def _():
                        wait_out(fslot, g + _AHEAD - _NBUF)
                else:
                    wait_out(fslot, g + _AHEAD - _NBUF)
                if b2 < _IBUF - _AHEAD:
                    fire(fslot, (b2 + _AHEAD) % _IBUF, g + _AHEAD)
                else:
                    @pl.when(go < n_outer - 1)
                    def _():
                        fire(fslot, (b2 + _AHEAD) % _IBUF, g + _AHEAD)
                drain(slot, b2)
                out_copy(slot, g)
            return ()

        lax.fori_loop(0, n_outer, outer, ())

        for i in range(tail):
            g_last = n_slots - tail + i
            wait_out(g_last % _NBUF, g_last)

    return k


def kernel(indices, table):
    b, h = indices.shape
    v, d = table.shape
    idx4 = (
        jnp.transpose(indices.astype(jnp.int32))
        .reshape(h // _TL, _TL, b // _TB, _TB)
        .transpose(0, 2, 1, 3)
    )
    out = _build(b, h, v, d)(idx4, table)
    return out[:, :, :, :d].reshape(b, h, d)
